# baseline (speedup 1.0000x reference)
import functools

import jax
import jax.numpy as jnp
from jax import lax
from jax.experimental import pallas as pl
from jax.experimental.pallas import tpu as pltpu
from jax.experimental.pallas import tpu_sc as plsc

N = 4194304
G = 30000
NC = 2
NS = 16
L = 16
NW = NC * NS
PER_W = N // NW

CHUNK = 4096
IN_RING = 4
OUT_RING = 2
N_CHUNKS = PER_W // CHUNK
PERIOD = max(IN_RING, OUT_RING)
assert PERIOD % IN_RING == 0 or IN_RING % PERIOD == 0
assert N_CHUNKS % PERIOD == 0 and PERIOD % OUT_RING == 0

LN2 = 0.6931471805599453
SQRT2 = 1.4142135623730951
MEAN_TOTAL_READS = 5000.0


def _vlog(x):
    bits = lax.bitcast_convert_type(x, jnp.int32)
    e = (bits >> 23) - 127
    m = lax.bitcast_convert_type((bits & 0x007FFFFF) | 0x3F800000,
                                 jnp.float32)
    big = m > SQRT2
    m = jnp.where(big, m * 0.5, m)
    ef = jnp.where(big, e + 1, e).astype(jnp.float32)
    s = (m - 1.0) / (m + 1.0)
    w = s * s
    p = s * (2.0 + w * (0.6666666666666667
                        + w * (0.4 + w * (0.2857142857142857
                                          + w * 0.2222222222222222))))
    return ef * LN2 + p


def _body(*refs):
    (idx_hbm, tor_hbm, dsr_hbm, tabA_hbm, tabB_hbm, consts_hbm,
     o0_hbm, o1_hbm, o2_hbm) = refs[:9]
    pos = 9
    tabA_v, tabB_v, consts_v = refs[pos:pos + 3]
    pos += 3
    ins = tuple(tuple(refs[pos + 3 * b:pos + 3 * b + 3])
                for b in range(IN_RING))
    pos += 3 * IN_RING
    outs = tuple(tuple(refs[pos + 3 * b:pos + 3 * b + 3])
                 for b in range(OUT_RING))
    pos += 3 * OUT_RING
    sins = refs[pos:pos + IN_RING]
    pos += IN_RING
    souts = refs[pos:pos + OUT_RING]
    pos += OUT_RING
    stab = refs[pos]

    cid = lax.axis_index("c")
    sid = lax.axis_index("s")
    wid = sid * NC + cid
    base = wid * PER_W

    in_hbm = (idx_hbm, tor_hbm, dsr_hbm)
    out_hbm = (o0_hbm, o1_hbm, o2_hbm)
    tabs_hbm = (tabA_hbm, tabB_hbm)
    tabs_v = (tabA_v, tabB_v)

    def issue_in(ci, b):
        off = base + ci * CHUNK
        for j in range(3):
            pltpu.async_copy(in_hbm[j].at[pl.ds(off, CHUNK)], ins[b][j],
                             sins[b])

    def wait_in(b):
        for j in range(3):
            pltpu.make_async_copy(in_hbm[j].at[pl.ds(0, CHUNK)], ins[b][j],
                                  sins[b]).wait()

    def issue_out(ci, b):
        off = base + ci * CHUNK
        for j in range(3):
            pltpu.async_copy(outs[b][j], out_hbm[j].at[pl.ds(off, CHUNK)],
                             souts[b])

    def wait_out(b):
        for j in range(3):
            pltpu.make_async_copy(outs[b][j], out_hbm[j].at[pl.ds(0, CHUNK)],
                                  souts[b]).wait()

    for b in range(IN_RING):
        issue_in(b, b)
    for j in range(2):
        pltpu.async_copy(tabs_hbm[j], tabs_v[j], stab)
    pltpu.sync_copy(consts_hbm, consts_v)
    ca = consts_v[pl.ds(0, L)]
    cb = consts_v[pl.ds(L, L)]
    for j in range(2):
        pltpu.make_async_copy(tabs_hbm[j], tabs_v[j], stab).wait()

    def gbody(g, carry):
        for k in range(PERIOD):
            ci = g * PERIOD + k
            bi = k % IN_RING
            bo = k % OUT_RING
            wait_in(bi)

            @pl.when(ci >= OUT_RING)
            def _():
                wait_out(bo)

            idx_v, tor_v, dsr_v = ins[bi]
            o0_v, o1_v, o2_v = outs[bo]

            def vec_body(vi, c2):
                sl = pl.ds(vi * L, L)
                gi = idx_v[sl]
                r = tor_v[sl] / dsr_v[sl]
                pf = _vlog(ca + cb * r)
                qa = plsc.load_gather(tabA_v, [gi])
                qb = plsc.load_gather(tabB_v, [gi >> 1])
                v0 = lax.bitcast_convert_type(qa & jnp.int32(-65536),
                                              jnp.float32)
                v1 = lax.bitcast_convert_type(qa << 16, jnp.float32)
                odd = (gi & 1) == 1
                v2 = lax.bitcast_convert_type(
                    jnp.where(odd, qb << 16, qb & jnp.int32(-65536)),
                    jnp.float32)
                o0_v[sl] = pf + v0
                o1_v[sl] = v1
                o2_v[sl] = v2
                return c2

            lax.fori_loop(0, CHUNK // L, vec_body, 0)
            issue_out(ci, bo)

            @pl.when(ci + IN_RING < N_CHUNKS)
            def _():
                issue_in(ci + IN_RING, bi)

        return carry

    lax.fori_loop(0, N_CHUNKS // PERIOD, gbody, 0)
    for b in range(OUT_RING):
        wait_out(b)


_scratch = (
    [pltpu.VMEM((G,), jnp.int32),
     pltpu.VMEM((G // 2,), jnp.int32),
     pltpu.VMEM((2 * L,), jnp.float32)]
    + [pltpu.VMEM((CHUNK,), jnp.int32) if j == 0 else
       pltpu.VMEM((CHUNK,), jnp.float32)
       for _ in range(IN_RING) for j in range(3)]
    + [pltpu.VMEM((CHUNK,), jnp.float32)
       for _ in range(OUT_RING) for _ in range(3)]
    + [pltpu.SemaphoreType.DMA] * (IN_RING + OUT_RING + 1)
)

_sc_call = functools.partial(
    pl.kernel,
    out_type=[jax.ShapeDtypeStruct((N,), jnp.float32)] * 3,
    mesh=plsc.VectorSubcoreMesh(core_axis_name="c", subcore_axis_name="s"),
    compiler_params=pltpu.CompilerParams(needs_layout_passes=False),
    scratch_types=_scratch,
)(_body)


def kernel(gene_index_tensor_n, cell_index_tensor_n, cell_features_nf,
           total_obs_reads_per_cell_tensor_n, downsampling_rate_tensor_n,
           global_prior_params_gr, logit_cell_size_beta):
    del cell_index_tensor_n, cell_features_nf
    beta = logit_cell_size_beta[0]
    ca = jax.nn.sigmoid(-beta)
    cb = jax.nn.sigmoid(beta) / MEAN_TOTAL_READS
    consts = jnp.concatenate([
        jnp.full((L,), ca, jnp.float32),
        jnp.full((L,), cb, jnp.float32),
    ])
    idx = gene_index_tensor_n.astype(jnp.int32)
    bb = lax.bitcast_convert_type(
        global_prior_params_gr.astype(jnp.bfloat16),
        jnp.uint16).astype(jnp.uint32)
    b0, b1, b2 = bb[:, 0], bb[:, 1], bb[:, 2]
    tabA = lax.bitcast_convert_type((b0 << 16) | b1, jnp.int32)
    tabB = lax.bitcast_convert_type((b2[0::2] << 16) | b2[1::2], jnp.int32)
    o0, o1, o2 = _sc_call(
        idx, total_obs_reads_per_cell_tensor_n, downsampling_rate_tensor_n,
        tabA, tabB, consts)
    return o0, o1, o2


# speedup vs baseline: 1.1954x; 1.1954x over previous
import functools

import jax
import jax.numpy as jnp
from jax import lax
from jax.experimental import pallas as pl
from jax.experimental.pallas import tpu as pltpu
from jax.experimental.pallas import tpu_sc as plsc

N = 4194304
G = 30000
NC = 2
NS = 16
L = 16
NW = NC * NS
PER_W = N // NW

CHUNK = 4096
IN_RING = 2
OUT_RING = 2
N_CHUNKS = PER_W // CHUNK
PERIOD = max(IN_RING, OUT_RING)
assert PERIOD % IN_RING == 0 or IN_RING % PERIOD == 0
assert N_CHUNKS % PERIOD == 0 and PERIOD % OUT_RING == 0

LN2 = 0.6931471805599453
SQRT2 = 1.4142135623730951
MEAN_TOTAL_READS = 5000.0


def _vlog(x):
    bits = lax.bitcast_convert_type(x, jnp.int32)
    e = (bits - 0x3F3504F3) >> 23
    m = lax.bitcast_convert_type(bits - (e << 23), jnp.float32)
    ef = e.astype(jnp.float32)
    s = (m - 1.0) / (m + 1.0)
    w = s * s
    p = s * (2.0 + w * (0.6666666666666667 + w * 0.4))
    return ef * LN2 + p


def _body(*refs):
    (idx_hbm, tor_hbm, dsr_hbm, tab0_hbm, tabp_hbm, consts_hbm,
     o0_hbm, o1_hbm, o2_hbm) = refs[:9]
    pos = 9
    tab0_v, tabp_v, consts_v = refs[pos:pos + 3]
    pos += 3
    ins = tuple(tuple(refs[pos + 3 * b:pos + 3 * b + 3])
                for b in range(IN_RING))
    pos += 3 * IN_RING
    outs = tuple(tuple(refs[pos + 3 * b:pos + 3 * b + 3])
                 for b in range(OUT_RING))
    pos += 3 * OUT_RING
    sins = refs[pos:pos + IN_RING]
    pos += IN_RING
    souts = refs[pos:pos + OUT_RING]
    pos += OUT_RING
    stab = refs[pos]

    cid = lax.axis_index("c")
    sid = lax.axis_index("s")
    wid = sid * NC + cid
    base = wid * PER_W

    in_hbm = (idx_hbm, tor_hbm, dsr_hbm)
    out_hbm = (o0_hbm, o1_hbm, o2_hbm)
    tabs_hbm = (tab0_hbm, tabp_hbm)
    tabs_v = (tab0_v, tabp_v)

    def issue_in(ci, b):
        off = base + ci * CHUNK
        for j in range(3):
            pltpu.async_copy(in_hbm[j].at[pl.ds(off, CHUNK)], ins[b][j],
                             sins[b])

    def wait_in(b):
        for j in range(3):
            pltpu.make_async_copy(in_hbm[j].at[pl.ds(0, CHUNK)], ins[b][j],
                                  sins[b]).wait()

    def issue_out(ci, b):
        off = base + ci * CHUNK
        for j in range(3):
            pltpu.async_copy(outs[b][j], out_hbm[j].at[pl.ds(off, CHUNK)],
                             souts[b])

    def wait_out(b):
        for j in range(3):
            pltpu.make_async_copy(outs[b][j], out_hbm[j].at[pl.ds(0, CHUNK)],
                                  souts[b]).wait()

    for b in range(IN_RING):
        issue_in(b, b)
    for j in range(2):
        pltpu.async_copy(tabs_hbm[j], tabs_v[j], stab)
    pltpu.sync_copy(consts_hbm, consts_v)
    ca = consts_v[pl.ds(0, L)]
    cb = consts_v[pl.ds(L, L)]
    for j in range(2):
        pltpu.make_async_copy(tabs_hbm[j], tabs_v[j], stab).wait()

    def gbody(g, carry):
        for k in range(PERIOD):
            ci = g * PERIOD + k
            bi = k % IN_RING
            bo = k % OUT_RING
            wait_in(bi)

            @pl.when(ci >= OUT_RING)
            def _():
                wait_out(bo)

            idx_v, tor_v, dsr_v = ins[bi]
            o0_v, o1_v, o2_v = outs[bo]

            def vec_body(vi, c2):
                sl = pl.ds(vi * L, L)
                gi = idx_v[sl]
                r = tor_v[sl] / dsr_v[sl]
                pf = _vlog(ca + cb * r)
                v0 = plsc.load_gather(tab0_v, [gi])
                pp = plsc.load_gather(tabp_v, [gi])
                v1 = lax.bitcast_convert_type(pp & jnp.int32(-65536),
                                              jnp.float32)
                v2 = lax.bitcast_convert_type(pp << 16, jnp.float32)
                o0_v[sl] = pf + v0
                o1_v[sl] = v1
                o2_v[sl] = v2
                return c2

            lax.fori_loop(0, CHUNK // L, vec_body, 0)
            issue_out(ci, bo)

            @pl.when(ci + IN_RING < N_CHUNKS)
            def _():
                issue_in(ci + IN_RING, bi)

        return carry

    lax.fori_loop(0, N_CHUNKS // PERIOD, gbody, 0)
    for b in range(OUT_RING):
        wait_out(b)


_scratch = (
    [pltpu.VMEM((G,), jnp.float32),
     pltpu.VMEM((G,), jnp.int32),
     pltpu.VMEM((2 * L,), jnp.float32)]
    + [pltpu.VMEM((CHUNK,), jnp.int32) if j == 0 else
       pltpu.VMEM((CHUNK,), jnp.float32)
       for _ in range(IN_RING) for j in range(3)]
    + [pltpu.VMEM((CHUNK,), jnp.float32)
       for _ in range(OUT_RING) for _ in range(3)]
    + [pltpu.SemaphoreType.DMA] * (IN_RING + OUT_RING + 1)
)

_sc_call = functools.partial(
    pl.kernel,
    out_type=[jax.ShapeDtypeStruct((N,), jnp.float32)] * 3,
    mesh=plsc.VectorSubcoreMesh(core_axis_name="c", subcore_axis_name="s"),
    compiler_params=pltpu.CompilerParams(needs_layout_passes=False),
    scratch_types=_scratch,
)(_body)


def kernel(gene_index_tensor_n, cell_index_tensor_n, cell_features_nf,
           total_obs_reads_per_cell_tensor_n, downsampling_rate_tensor_n,
           global_prior_params_gr, logit_cell_size_beta):
    del cell_index_tensor_n, cell_features_nf
    beta = logit_cell_size_beta[0]
    ca = jax.nn.sigmoid(-beta)
    cb = jax.nn.sigmoid(beta) / MEAN_TOTAL_READS
    consts = jnp.concatenate([
        jnp.full((L,), ca, jnp.float32),
        jnp.full((L,), cb, jnp.float32),
    ])
    idx = gene_index_tensor_n.astype(jnp.int32)
    tab0 = global_prior_params_gr[:, 0]
    b1 = lax.bitcast_convert_type(
        global_prior_params_gr[:, 1].astype(jnp.bfloat16),
        jnp.uint16).astype(jnp.uint32)
    b2 = lax.bitcast_convert_type(
        global_prior_params_gr[:, 2].astype(jnp.bfloat16),
        jnp.uint16).astype(jnp.uint32)
    tabp = lax.bitcast_convert_type((b1 << 16) | b2, jnp.int32)
    o0, o1, o2 = _sc_call(
        idx, total_obs_reads_per_cell_tensor_n, downsampling_rate_tensor_n,
        tab0, tabp, consts)
    return o0, o1, o2


# speedup vs baseline: 1.2006x; 1.0044x over previous
import functools

import jax
import jax.numpy as jnp
from jax import lax
from jax.experimental import pallas as pl
from jax.experimental.pallas import tpu as pltpu
from jax.experimental.pallas import tpu_sc as plsc

N = 4194304
G = 30000
NC = 2
NS = 16
L = 16
NW = NC * NS
PER_W = N // NW

CHUNK = 4096
IN_RING = 2
OUT_RING = 2
N_CHUNKS = PER_W // CHUNK
PERIOD = max(IN_RING, OUT_RING)
assert PERIOD % IN_RING == 0 or IN_RING % PERIOD == 0
assert N_CHUNKS % PERIOD == 0 and PERIOD % OUT_RING == 0

LN2 = 0.6931471805599453
SQRT2 = 1.4142135623730951
MEAN_TOTAL_READS = 5000.0


def _vlog(x):
    bits = lax.bitcast_convert_type(x, jnp.int32)
    e = (bits - 0x3F3504F3) >> 23
    m = lax.bitcast_convert_type(bits - (e << 23), jnp.float32)
    ef = e.astype(jnp.float32)
    s = (m - 1.0) / (m + 1.0)
    w = s * s
    p = s * (2.0 + w * (0.6666666666666667 + w * 0.4))
    return ef * LN2 + p


def _body(*refs):
    (idx_hbm, tor_hbm, dsr_hbm, tab0_hbm, tabp_hbm, consts_hbm,
     o0_hbm, o1_hbm, o2_hbm) = refs[:9]
    pos = 9
    tab0_v, tabp_v, consts_v = refs[pos:pos + 3]
    pos += 3
    ins = tuple(tuple(refs[pos + 3 * b:pos + 3 * b + 3])
                for b in range(IN_RING))
    pos += 3 * IN_RING
    outs = tuple(tuple(refs[pos + 3 * b:pos + 3 * b + 3])
                 for b in range(OUT_RING))
    pos += 3 * OUT_RING
    sins = refs[pos:pos + IN_RING]
    pos += IN_RING
    souts = refs[pos:pos + OUT_RING]
    pos += OUT_RING
    stab = refs[pos]

    cid = lax.axis_index("c")
    sid = lax.axis_index("s")
    wid = sid * NC + cid
    base = wid * PER_W

    in_hbm = (idx_hbm, tor_hbm, dsr_hbm)
    out_hbm = (o0_hbm, o1_hbm, o2_hbm)
    tabs_hbm = (tab0_hbm, tabp_hbm)
    tabs_v = (tab0_v, tabp_v)

    def issue_in(ci, b):
        off = base + ci * CHUNK
        for j in range(3):
            pltpu.async_copy(in_hbm[j].at[pl.ds(off, CHUNK)], ins[b][j],
                             sins[b])

    def wait_in(b):
        for j in range(3):
            pltpu.make_async_copy(in_hbm[j].at[pl.ds(0, CHUNK)], ins[b][j],
                                  sins[b]).wait()

    def issue_out(ci, b):
        off = base + ci * CHUNK
        for j in range(3):
            pltpu.async_copy(outs[b][j], out_hbm[j].at[pl.ds(off, CHUNK)],
                             souts[b])

    def wait_out(b):
        for j in range(3):
            pltpu.make_async_copy(outs[b][j], out_hbm[j].at[pl.ds(0, CHUNK)],
                                  souts[b]).wait()

    for b in range(IN_RING):
        issue_in(b, b)
    for j in range(2):
        pltpu.async_copy(tabs_hbm[j], tabs_v[j], stab)
    pltpu.sync_copy(consts_hbm, consts_v)
    ca = consts_v[pl.ds(0, L)]
    cb = consts_v[pl.ds(L, L)]
    for j in range(2):
        pltpu.make_async_copy(tabs_hbm[j], tabs_v[j], stab).wait()

    def gbody(g, carry):
        for k in range(PERIOD):
            ci = g * PERIOD + k
            bi = k % IN_RING
            bo = k % OUT_RING
            wait_in(bi)

            @pl.when(ci >= OUT_RING)
            def _():
                wait_out(bo)

            idx_v, tor_v, dsr_v = ins[bi]
            o0_v, o1_v, o2_v = outs[bo]

            def vec_body(vi, c2):
                sl = pl.ds(vi * L, L)
                gi = idx_v[sl]
                r = tor_v[sl] / dsr_v[sl]
                pf = _vlog(ca + cb * r)
                v0 = plsc.load_gather(tab0_v, [gi])
                pp = plsc.load_gather(tabp_v, [gi])
                v1 = lax.bitcast_convert_type(pp & jnp.int32(-65536),
                                              jnp.float32)
                v2 = lax.bitcast_convert_type(pp << 16, jnp.float32)
                o0_v[sl] = pf + v0
                o1_v[sl] = v1
                o2_v[sl] = v2
                return c2

            lax.fori_loop(0, CHUNK // L, vec_body, 0)

            @pl.when(ci + IN_RING < N_CHUNKS)
            def _():
                issue_in(ci + IN_RING, bi)

            issue_out(ci, bo)

        return carry

    lax.fori_loop(0, N_CHUNKS // PERIOD, gbody, 0)
    for b in range(OUT_RING):
        wait_out(b)


_scratch = (
    [pltpu.VMEM((G,), jnp.float32),
     pltpu.VMEM((G,), jnp.int32),
     pltpu.VMEM((2 * L,), jnp.float32)]
    + [pltpu.VMEM((CHUNK,), jnp.int32) if j == 0 else
       pltpu.VMEM((CHUNK,), jnp.float32)
       for _ in range(IN_RING) for j in range(3)]
    + [pltpu.VMEM((CHUNK,), jnp.float32)
       for _ in range(OUT_RING) for _ in range(3)]
    + [pltpu.SemaphoreType.DMA] * (IN_RING + OUT_RING + 1)
)

_sc_call = functools.partial(
    pl.kernel,
    out_type=[jax.ShapeDtypeStruct((N,), jnp.float32)] * 3,
    mesh=plsc.VectorSubcoreMesh(core_axis_name="c", subcore_axis_name="s"),
    compiler_params=pltpu.CompilerParams(needs_layout_passes=False),
    scratch_types=_scratch,
)(_body)


def kernel(gene_index_tensor_n, cell_index_tensor_n, cell_features_nf,
           total_obs_reads_per_cell_tensor_n, downsampling_rate_tensor_n,
           global_prior_params_gr, logit_cell_size_beta):
    del cell_index_tensor_n, cell_features_nf
    beta = logit_cell_size_beta[0]
    ca = jax.nn.sigmoid(-beta)
    cb = jax.nn.sigmoid(beta) / MEAN_TOTAL_READS
    consts = jnp.concatenate([
        jnp.full((L,), ca, jnp.float32),
        jnp.full((L,), cb, jnp.float32),
    ])
    idx = gene_index_tensor_n.astype(jnp.int32)
    tab0 = global_prior_params_gr[:, 0]
    b1 = lax.bitcast_convert_type(
        global_prior_params_gr[:, 1].astype(jnp.bfloat16),
        jnp.uint16).astype(jnp.uint32)
    b2 = lax.bitcast_convert_type(
        global_prior_params_gr[:, 2].astype(jnp.bfloat16),
        jnp.uint16).astype(jnp.uint32)
    tabp = lax.bitcast_convert_type((b1 << 16) | b2, jnp.int32)
    o0, o1, o2 = _sc_call(
        idx, total_obs_reads_per_cell_tensor_n, downsampling_rate_tensor_n,
        tab0, tabp, consts)
    return o0, o1, o2
